# baseline (speedup 1.0000x reference)
import functools

import jax
import jax.numpy as jnp
from jax import lax
from jax.experimental import pallas as pl
from jax.experimental.pallas import tpu as pltpu
from jax.experimental.pallas import tpu_sc as plsc

B = 16384
N = 20
D = 32
V = 1000000
RPB = 4
VB = V // RPB
NC = 2
NS = 16
NW = NC * NS
BPW = B // NW
C = 32
G = C // 16
NIDX = C * N


def _log_sigmoid(x):
    z = jnp.exp(-jnp.abs(x))
    t = z / (z + 2.0)
    t2 = t * t
    p = 2.0 * t * (1.0 + t2 * (1.0 / 3.0 + t2 * (0.2 + t2 * (1.0 / 7.0 + t2 * (1.0 / 9.0)))))
    return jnp.minimum(x, 0.0) - p


@functools.partial(
    pl.kernel,
    mesh=plsc.VectorSubcoreMesh(core_axis_name="c", subcore_axis_name="s"),
    compiler_params=pltpu.CompilerParams(needs_layout_passes=False),
    out_type=jax.ShapeDtypeStruct((B,), jnp.float32),
    scratch_types=[
        pltpu.VMEM((C,), jnp.int32),
        pltpu.VMEM((C,), jnp.int32),
        pltpu.VMEM((NIDX,), jnp.int32),
        pltpu.VMEM((C,), jnp.int32),
        pltpu.VMEM((C,), jnp.int32),
        pltpu.VMEM((NIDX,), jnp.int32),
        pltpu.VMEM((C, 128), jnp.float32),
        pltpu.VMEM((C, 128), jnp.float32),
        pltpu.VMEM((NIDX, 128), jnp.float32),
        pltpu.VMEM((C,), jnp.float32),
        pltpu.SemaphoreType.DMA,
    ],
)
def _sc_loss(center_hbm, outside_hbm, negflat_hbm, vtab_hbm, utab_hbm,
             out_hbm, cidx_v, oidx_v, nidx_v, cblk_v, oblk_v, nblk_v,
             vblocks_v, oblocks_v, nblocks_v, out_v, sem):
    wid = lax.axis_index("s") * NC + lax.axis_index("c")
    base = wid * BPW

    def chunk_body(c, carry):
        start = base + c * C
        pltpu.sync_copy(center_hbm.at[pl.ds(start, C)], cidx_v)
        pltpu.sync_copy(outside_hbm.at[pl.ds(start, C)], oidx_v)
        pltpu.sync_copy(negflat_hbm.at[pl.ds(start * N, NIDX)], nidx_v)
        for i in range(C // 16):
            cblk_v[pl.ds(i * 16, 16)] = cidx_v[pl.ds(i * 16, 16)] >> 2
            oblk_v[pl.ds(i * 16, 16)] = oidx_v[pl.ds(i * 16, 16)] >> 2
        for i in range(NIDX // 16):
            nblk_v[pl.ds(i * 16, 16)] = nidx_v[pl.ds(i * 16, 16)] >> 2
        copies = [
            pltpu.async_copy(vtab_hbm.at[cblk_v], vblocks_v, sem),
            pltpu.async_copy(utab_hbm.at[oblk_v], oblocks_v, sem),
        ]
        for j in range(NIDX // 128):
            copies.append(pltpu.async_copy(
                utab_hbm.at[nblk_v.at[pl.ds(j * 128, 128)]],
                nblocks_v.at[pl.ds(j * 128, 128)], sem))
        for cp in copies:
            cp.wait()

        def group_body(g, gcarry):
            zeros = jnp.zeros((16,), jnp.float32)
            lanes = lax.iota(jnp.int32, 16)
            rows16 = g * 16 + lanes
            vcol = (cidx_v[pl.ds(g * 16, 16)] & 3) * D
            ocol = (oidx_v[pl.ds(g * 16, 16)] & 3) * D
            nrow = [rows16 * N + n for n in range(N)]
            nvec = [plsc.load_gather(nidx_v, [nrow[n]]) for n in range(N)]
            ncol = [(nvec[n] & 3) * D for n in range(N)]

            def d_body(dd, accs):
                v_d = plsc.load_gather(vblocks_v, [rows16, vcol + dd])
                o_d = plsc.load_gather(oblocks_v, [rows16, ocol + dd])
                new = [accs[0] + v_d * o_d]
                for n in range(N):
                    u_d = plsc.load_gather(nblocks_v, [nrow[n], ncol[n] + dd])
                    new.append(accs[1 + n] + v_d * u_d)
                return tuple(new)

            accs = lax.fori_loop(0, D, d_body, (zeros,) * (N + 1))
            loss = _log_sigmoid(jnp.clip(accs[0], -10.0, 10.0))
            for n in range(N):
                loss = loss + _log_sigmoid(-jnp.clip(accs[1 + n], -10.0, 10.0))
            out_v[pl.ds(g * 16, 16)] = -loss
            return gcarry

        lax.fori_loop(0, G, group_body, 0)
        pltpu.sync_copy(out_v, out_hbm.at[pl.ds(start, C)])
        return carry

    lax.fori_loop(0, BPW // C, chunk_body, 0)


def kernel(center, outside, negative, emb_v, emb_u):
    vtab = emb_v.reshape(VB, RPB * D)
    utab = emb_u.reshape(VB, RPB * D)
    return _sc_loss(center, outside, negative.reshape(-1), vtab, utab)


# speedup vs baseline: 1.4396x; 1.4396x over previous
import functools

import jax
import jax.numpy as jnp
from jax import lax
from jax.experimental import pallas as pl
from jax.experimental.pallas import tpu as pltpu
from jax.experimental.pallas import tpu_sc as plsc

B = 16384
N = 20
D = 32
V = 1000000
RPB = 4
VB = V // RPB
NC = 2
NS = 16
NW = NC * NS
BPW = B // NW
C = 32
G = C // 16
NIDX = C * N


def _log_sigmoid(x):
    z = jnp.exp(-jnp.abs(x))
    t = z / (z + 2.0)
    t2 = t * t
    p = 2.0 * t * (1.0 + t2 * (1.0 / 3.0 + t2 * (0.2 + t2 * (1.0 / 7.0 + t2 * (1.0 / 9.0)))))
    return jnp.minimum(x, 0.0) - p


@functools.partial(
    pl.kernel,
    mesh=plsc.VectorSubcoreMesh(core_axis_name="c", subcore_axis_name="s"),
    compiler_params=pltpu.CompilerParams(needs_layout_passes=False),
    out_type=jax.ShapeDtypeStruct((B,), jnp.float32),
    scratch_types=[
        pltpu.VMEM((C,), jnp.int32),
        pltpu.VMEM((C,), jnp.int32),
        pltpu.VMEM((NIDX,), jnp.int32),
        pltpu.VMEM((C,), jnp.int32),
        pltpu.VMEM((C,), jnp.int32),
        pltpu.VMEM((NIDX,), jnp.int32),
        pltpu.VMEM((C, 128), jnp.float32),
        pltpu.VMEM((C, 128), jnp.float32),
        pltpu.VMEM((NIDX, 128), jnp.float32),
        pltpu.VMEM((C,), jnp.float32),
        pltpu.SemaphoreType.DMA,
    ],
)
def _sc_loss(center_hbm, outside_hbm, negflat_hbm, vtab_hbm, utab_hbm,
             out_hbm, cidx_v, oidx_v, nidx_v, cblk_v, oblk_v, nblk_v,
             vblocks_v, oblocks_v, nblocks_v, out_v, sem):
    wid = lax.axis_index("s") * NC + lax.axis_index("c")
    base = wid * BPW

    def chunk_body(c, carry):
        start = base + c * C
        pltpu.sync_copy(center_hbm.at[pl.ds(start, C)], cidx_v)
        pltpu.sync_copy(outside_hbm.at[pl.ds(start, C)], oidx_v)
        pltpu.sync_copy(negflat_hbm.at[pl.ds(start * N, NIDX)], nidx_v)
        def _blk(r):
            return ((r >> 13) << 11) | (r & 2047)

        for i in range(C // 16):
            cblk_v[pl.ds(i * 16, 16)] = _blk(cidx_v[pl.ds(i * 16, 16)])
            oblk_v[pl.ds(i * 16, 16)] = _blk(oidx_v[pl.ds(i * 16, 16)])
        for i in range(NIDX // 16):
            nblk_v[pl.ds(i * 16, 16)] = _blk(nidx_v[pl.ds(i * 16, 16)])
        copies = [
            pltpu.async_copy(vtab_hbm.at[cblk_v], vblocks_v, sem),
            pltpu.async_copy(utab_hbm.at[oblk_v], oblocks_v, sem),
        ]
        for j in range(NIDX // 128):
            copies.append(pltpu.async_copy(
                utab_hbm.at[nblk_v.at[pl.ds(j * 128, 128)]],
                nblocks_v.at[pl.ds(j * 128, 128)], sem))
        for cp in copies:
            cp.wait()

        def group_body(g, gcarry):
            zeros = jnp.zeros((16,), jnp.float32)
            lanes = lax.iota(jnp.int32, 16)
            rows16 = g * 16 + lanes
            vcol = ((cidx_v[pl.ds(g * 16, 16)] >> 11) & 3) * D
            ocol = ((oidx_v[pl.ds(g * 16, 16)] >> 11) & 3) * D
            nrow = [rows16 * N + n for n in range(N)]
            nvec = [plsc.load_gather(nidx_v, [nrow[n]]) for n in range(N)]
            ncol = [((nvec[n] >> 11) & 3) * D for n in range(N)]

            def d_body(dd, accs):
                v_d = plsc.load_gather(vblocks_v, [rows16, vcol + dd])
                o_d = plsc.load_gather(oblocks_v, [rows16, ocol + dd])
                new = [accs[0] + v_d * o_d]
                for n in range(N):
                    u_d = plsc.load_gather(nblocks_v, [nrow[n], ncol[n] + dd])
                    new.append(accs[1 + n] + v_d * u_d)
                return tuple(new)

            accs = lax.fori_loop(0, D, d_body, (zeros,) * (N + 1))
            loss = _log_sigmoid(jnp.clip(accs[0], -10.0, 10.0))
            for n in range(N):
                loss = loss + _log_sigmoid(-jnp.clip(accs[1 + n], -10.0, 10.0))
            out_v[pl.ds(g * 16, 16)] = -loss
            return gcarry

        lax.fori_loop(0, G, group_body, 0)
        pltpu.sync_copy(out_v, out_hbm.at[pl.ds(start, C)])
        return carry

    lax.fori_loop(0, BPW // C, chunk_body, 0)


TW = 8192
PW = TW // RPB
TGRID = (V + TW - 1) // TW
VBP = TGRID * PW


def _tr_body(x_ref, o_ref):
    xt = x_ref[...].T
    o_ref[...] = jnp.concatenate(
        [xt[j * PW:(j + 1) * PW, :] for j in range(RPB)], axis=1)


def _to_blocks(emb):
    return pl.pallas_call(
        _tr_body,
        grid=(TGRID,),
        in_specs=[pl.BlockSpec((D, TW), lambda i: (0, i))],
        out_specs=pl.BlockSpec((PW, RPB * D), lambda i: (i, 0)),
        out_shape=jax.ShapeDtypeStruct((VBP, RPB * D), jnp.float32),
    )(emb.T)


def kernel(center, outside, negative, emb_v, emb_u):
    return _sc_loss(center, outside, negative.reshape(-1),
                    _to_blocks(emb_v), _to_blocks(emb_u))


# speedup vs baseline: 1.5544x; 1.0798x over previous
import functools

import jax
import jax.numpy as jnp
from jax import lax
from jax.experimental import pallas as pl
from jax.experimental.pallas import tpu as pltpu
from jax.experimental.pallas import tpu_sc as plsc

B = 16384
N = 20
D = 32
V = 1000000
RPB = 4
VB = V // RPB
NC = 2
NS = 16
NW = NC * NS
BPW = B // NW
C = 128
G = C // 16
NIDX = C * N


def _log_sigmoid(x):
    z = jnp.exp(-jnp.abs(x))
    t = z / (z + 2.0)
    t2 = t * t
    p = 2.0 * t * (1.0 + t2 * (1.0 / 3.0 + t2 * (0.2 + t2 * (1.0 / 7.0 + t2 * (1.0 / 9.0)))))
    return jnp.minimum(x, 0.0) - p


@functools.partial(
    pl.kernel,
    mesh=plsc.VectorSubcoreMesh(core_axis_name="c", subcore_axis_name="s"),
    compiler_params=pltpu.CompilerParams(
        needs_layout_passes=False, use_tc_tiling_on_sc=False),
    out_type=jax.ShapeDtypeStruct((B,), jnp.float32),
    scratch_types=[
        pltpu.VMEM((C,), jnp.int32),
        pltpu.VMEM((C,), jnp.int32),
        pltpu.VMEM((NIDX,), jnp.int32),
        pltpu.VMEM((C,), jnp.int32),
        pltpu.VMEM((C,), jnp.int32),
        pltpu.VMEM((NIDX,), jnp.int32),
        pltpu.VMEM((C, D), jnp.float32),
        pltpu.VMEM((C, D), jnp.float32),
        pltpu.VMEM((NIDX, D), jnp.float32),
        pltpu.VMEM((C,), jnp.float32),
        pltpu.SemaphoreType.DMA,
    ],
)
def _sc_loss(center_hbm, outside_hbm, negflat_hbm, vtab_hbm, utab_hbm,
             out_hbm, cidx_v, oidx_v, nidx_v, cblk_v, oblk_v, nblk_v,
             vblocks_v, oblocks_v, nblocks_v, out_v, sem):
    wid = lax.axis_index("s") * NC + lax.axis_index("c")
    base = wid * BPW

    def chunk_body(c, carry):
        start = base + c * C
        pltpu.sync_copy(center_hbm.at[pl.ds(start, C)], cidx_v)
        pltpu.sync_copy(outside_hbm.at[pl.ds(start, C)], oidx_v)
        pltpu.sync_copy(negflat_hbm.at[pl.ds(start * N, NIDX)], nidx_v)
        def _blk(r):
            return ((r >> 13) << 13) | ((r & 2047) << 2) | ((r >> 11) & 3)

        for i in range(C // 16):
            cblk_v[pl.ds(i * 16, 16)] = _blk(cidx_v[pl.ds(i * 16, 16)])
            oblk_v[pl.ds(i * 16, 16)] = _blk(oidx_v[pl.ds(i * 16, 16)])
        for i in range(NIDX // 16):
            nblk_v[pl.ds(i * 16, 16)] = _blk(nidx_v[pl.ds(i * 16, 16)])
        copies = [
            pltpu.async_copy(vtab_hbm.at[cblk_v], vblocks_v, sem),
            pltpu.async_copy(utab_hbm.at[oblk_v], oblocks_v, sem),
        ]
        for j in range(NIDX // 128):
            copies.append(pltpu.async_copy(
                utab_hbm.at[nblk_v.at[pl.ds(j * 128, 128)]],
                nblocks_v.at[pl.ds(j * 128, 128)], sem))
        for cp in copies:
            cp.wait()

        def group_body(g, gcarry):
            zeros = jnp.zeros((16,), jnp.float32)
            lanes = lax.iota(jnp.int32, 16)
            rows16 = g * 16 + lanes
            nrow = [rows16 * N + n for n in range(N)]

            def d_body(dd, accs):
                col = jnp.full((16,), dd, dtype=jnp.int32)
                v_d = plsc.load_gather(vblocks_v, [rows16, col])
                o_d = plsc.load_gather(oblocks_v, [rows16, col])
                new = [accs[0] + v_d * o_d]
                for n in range(N):
                    u_d = plsc.load_gather(nblocks_v, [nrow[n], col])
                    new.append(accs[1 + n] + v_d * u_d)
                return tuple(new)

            accs = lax.fori_loop(0, D, d_body, (zeros,) * (N + 1))
            loss = _log_sigmoid(jnp.clip(accs[0], -10.0, 10.0))
            for n in range(N):
                loss = loss + _log_sigmoid(-jnp.clip(accs[1 + n], -10.0, 10.0))
            out_v[pl.ds(g * 16, 16)] = -loss
            return gcarry

        lax.fori_loop(0, G, group_body, 0)
        pltpu.sync_copy(out_v, out_hbm.at[pl.ds(start, C)])
        return carry

    lax.fori_loop(0, BPW // C, chunk_body, 0)


TW = 8192
PW = TW // RPB
TGRID = (V + TW - 1) // TW
VBP = TGRID * PW


def _tr_body(x_ref, o_ref):
    xt = x_ref[...].T
    o_ref[...] = jnp.concatenate(
        [xt[j * PW:(j + 1) * PW, :] for j in range(RPB)], axis=1)


def _to_blocks(emb):
    return pl.pallas_call(
        _tr_body,
        grid=(TGRID,),
        in_specs=[pl.BlockSpec((D, TW), lambda i: (0, i))],
        out_specs=pl.BlockSpec((PW, RPB * D), lambda i: (i, 0)),
        out_shape=jax.ShapeDtypeStruct((VBP, RPB * D), jnp.float32),
    )(emb.T)


def kernel(center, outside, negative, emb_v, emb_u):
    vtab = _to_blocks(emb_v).reshape(VBP * RPB, D)
    utab = _to_blocks(emb_u).reshape(VBP * RPB, D)
    return _sc_loss(center, outside, negative.reshape(-1), vtab, utab)


# speedup vs baseline: 2.0614x; 1.3262x over previous
import functools

import jax
import jax.numpy as jnp
from jax import lax
from jax.experimental import pallas as pl
from jax.experimental.pallas import tpu as pltpu
from jax.experimental.pallas import tpu_sc as plsc

B = 16384
N = 20
D = 32
V = 1000000
RPB = 4
VB = V // RPB
NC = 2
NS = 16
NW = NC * NS
BPW = B // NW
C = 128
G = C // 16
NIDX = C * N


def _log_sigmoid(x):
    z = jnp.exp(-jnp.abs(x))
    t = z / (z + 2.0)
    t2 = t * t
    p = 2.0 * t * (1.0 + t2 * (1.0 / 3.0 + t2 * (0.2 + t2 * (1.0 / 7.0 + t2 * (1.0 / 9.0)))))
    return jnp.minimum(x, 0.0) - p


@functools.partial(
    pl.kernel,
    mesh=plsc.VectorSubcoreMesh(core_axis_name="c", subcore_axis_name="s"),
    compiler_params=pltpu.CompilerParams(
        needs_layout_passes=False, use_tc_tiling_on_sc=False),
    out_type=jax.ShapeDtypeStruct((B,), jnp.float32),
    scratch_types=[
        pltpu.VMEM((C,), jnp.int32),
        pltpu.VMEM((C,), jnp.int32),
        pltpu.VMEM((NIDX,), jnp.int32),
        pltpu.VMEM((C,), jnp.int32),
        pltpu.VMEM((C,), jnp.int32),
        pltpu.VMEM((NIDX,), jnp.int32),
        pltpu.VMEM((C, D), jnp.float32),
        pltpu.VMEM((C, D), jnp.float32),
        pltpu.VMEM((NIDX, D), jnp.float32),
        pltpu.VMEM((C,), jnp.float32),
        pltpu.SemaphoreType.DMA,
    ],
)
def _sc_loss(center_hbm, outside_hbm, negflat_hbm, vtab_hbm, utab_hbm,
             out_hbm, cidx_v, oidx_v, nidx_v, cblk_v, oblk_v, nblk_v,
             vblocks_v, oblocks_v, nblocks_v, out_v, sem):
    wid = lax.axis_index("s") * NC + lax.axis_index("c")
    base = wid * BPW

    def chunk_body(c, carry):
        start = base + c * C
        pltpu.sync_copy(center_hbm.at[pl.ds(start, C)], cidx_v)
        pltpu.sync_copy(outside_hbm.at[pl.ds(start, C)], oidx_v)
        pltpu.sync_copy(negflat_hbm.at[pl.ds(start * N, NIDX)], nidx_v)
        def _blk(r):
            return ((r >> 13) << 13) | ((r & 2047) << 2) | ((r >> 11) & 3)

        for i in range(C // 16):
            cblk_v[pl.ds(i * 16, 16)] = _blk(cidx_v[pl.ds(i * 16, 16)])
            oblk_v[pl.ds(i * 16, 16)] = _blk(oidx_v[pl.ds(i * 16, 16)])
        for i in range(NIDX // 16):
            nblk_v[pl.ds(i * 16, 16)] = _blk(nidx_v[pl.ds(i * 16, 16)])
        copies = [
            pltpu.async_copy(vtab_hbm.at[cblk_v], vblocks_v, sem),
            pltpu.async_copy(utab_hbm.at[oblk_v], oblocks_v, sem),
        ]
        for j in range(NIDX // 128):
            copies.append(pltpu.async_copy(
                utab_hbm.at[nblk_v.at[pl.ds(j * 128, 128)]],
                nblocks_v.at[pl.ds(j * 128, 128)], sem))
        for cp in copies:
            cp.wait()

        def group_body(g, gcarry):
            zeros = jnp.zeros((16,), jnp.float32)
            lanes = lax.iota(jnp.int32, 16)
            rows16 = g * 16 + lanes
            nrow = [rows16 * N + n for n in range(N)]

            def d_body(dd, accs):
                col = (dd + lanes) & (D - 1)
                v_d = plsc.load_gather(vblocks_v, [rows16, col])
                o_d = plsc.load_gather(oblocks_v, [rows16, col])
                new = [accs[0] + v_d * o_d]
                for n in range(N):
                    u_d = plsc.load_gather(nblocks_v, [nrow[n], col])
                    new.append(accs[1 + n] + v_d * u_d)
                return tuple(new)

            accs = lax.fori_loop(0, D, d_body, (zeros,) * (N + 1))
            loss = _log_sigmoid(jnp.clip(accs[0], -10.0, 10.0))
            for n in range(N):
                loss = loss + _log_sigmoid(-jnp.clip(accs[1 + n], -10.0, 10.0))
            out_v[pl.ds(g * 16, 16)] = -loss
            return gcarry

        lax.fori_loop(0, G, group_body, 0)
        pltpu.sync_copy(out_v, out_hbm.at[pl.ds(start, C)])
        return carry

    lax.fori_loop(0, BPW // C, chunk_body, 0)


TW = 8192
PW = TW // RPB
TGRID = (V + TW - 1) // TW
VBP = TGRID * PW


def _tr_body(x_ref, o_ref):
    xt = x_ref[...].T
    o_ref[...] = jnp.concatenate(
        [xt[j * PW:(j + 1) * PW, :] for j in range(RPB)], axis=1)


def _to_blocks(emb):
    return pl.pallas_call(
        _tr_body,
        grid=(TGRID,),
        in_specs=[pl.BlockSpec((D, TW), lambda i: (0, i))],
        out_specs=pl.BlockSpec((PW, RPB * D), lambda i: (i, 0)),
        out_shape=jax.ShapeDtypeStruct((VBP, RPB * D), jnp.float32),
    )(emb.T)


def kernel(center, outside, negative, emb_v, emb_u):
    vtab = _to_blocks(emb_v).reshape(VBP * RPB, D)
    utab = _to_blocks(emb_u).reshape(VBP * RPB, D)
    return _sc_loss(center, outside, negative.reshape(-1), vtab, utab)


# speedup vs baseline: 3.3331x; 1.6169x over previous
import functools

import jax
import jax.numpy as jnp
from jax import lax
from jax.experimental import pallas as pl
from jax.experimental.pallas import tpu as pltpu
from jax.experimental.pallas import tpu_sc as plsc

B = 16384
N = 20
D = 32
V = 1000000
RPB = 4
VB = V // RPB
NC = 2
NS = 16
NW = NC * NS
BPW = B // NW
C = 128
G = C // 16
NIDX = C * N


def _log_sigmoid(x):
    z = jnp.exp(-jnp.abs(x))
    t = z / (z + 2.0)
    t2 = t * t
    p = 2.0 * t * (1.0 + t2 * (1.0 / 3.0 + t2 * (0.2 + t2 * (1.0 / 7.0 + t2 * (1.0 / 9.0)))))
    return jnp.minimum(x, 0.0) - p


@functools.partial(
    pl.kernel,
    mesh=plsc.VectorSubcoreMesh(core_axis_name="c", subcore_axis_name="s"),
    compiler_params=pltpu.CompilerParams(
        needs_layout_passes=False, use_tc_tiling_on_sc=False),
    out_type=jax.ShapeDtypeStruct((B,), jnp.float32),
    scratch_types=[
        pltpu.VMEM((C,), jnp.int32),
        pltpu.VMEM((C,), jnp.int32),
        pltpu.VMEM((NIDX,), jnp.int32),
        pltpu.VMEM((C,), jnp.int32),
        pltpu.VMEM((C,), jnp.int32),
        pltpu.VMEM((NIDX,), jnp.int32),
        pltpu.VMEM((C, D), jnp.float32),
        pltpu.VMEM((C, D), jnp.float32),
        pltpu.VMEM((NIDX, D), jnp.float32),
        pltpu.VMEM((C,), jnp.float32),
        pltpu.SemaphoreType.DMA,
    ],
)
def _sc_loss(center_hbm, outside_hbm, negflat_hbm, vtab_hbm, utab_hbm,
             out_hbm, cidx_v, oidx_v, nidx_v, cblk_v, oblk_v, nblk_v,
             vblocks_v, oblocks_v, nblocks_v, out_v, sem):
    wid = lax.axis_index("s") * NC + lax.axis_index("c")
    base = wid * BPW

    def chunk_body(c, carry):
        start = base + c * C
        pltpu.sync_copy(center_hbm.at[pl.ds(start, C)], cidx_v)
        pltpu.sync_copy(outside_hbm.at[pl.ds(start, C)], oidx_v)
        pltpu.sync_copy(negflat_hbm.at[pl.ds(start * N, NIDX)], nidx_v)
        def _blk(r):
            return ((r >> 13) << 13) | ((r & 2047) << 2) | ((r >> 11) & 3)

        for i in range(C // 16):
            cblk_v[pl.ds(i * 16, 16)] = _blk(cidx_v[pl.ds(i * 16, 16)])
            oblk_v[pl.ds(i * 16, 16)] = _blk(oidx_v[pl.ds(i * 16, 16)])
        for i in range(NIDX // 16):
            nblk_v[pl.ds(i * 16, 16)] = _blk(nidx_v[pl.ds(i * 16, 16)])
        copies = [
            pltpu.async_copy(vtab_hbm.at[cblk_v], vblocks_v, sem),
            pltpu.async_copy(utab_hbm.at[oblk_v], oblocks_v, sem),
        ]
        for j in range(NIDX // 128):
            copies.append(pltpu.async_copy(
                utab_hbm.at[nblk_v.at[pl.ds(j * 128, 128)]],
                nblocks_v.at[pl.ds(j * 128, 128)], sem))
        for cp in copies:
            cp.wait()

        def group_body(g, gcarry):
            zeros = jnp.zeros((16,), jnp.float32)
            lanes = lax.iota(jnp.int32, 16)
            rows16 = g * 16 + lanes
            nrow = [rows16 * N + n for n in range(N)]

            def d_body(dd, accs):
                col = (dd + lanes) & (D - 1)
                v_d = plsc.load_gather(vblocks_v, [rows16, col])
                o_d = plsc.load_gather(oblocks_v, [rows16, col])
                new = [accs[0] + v_d * o_d]
                for n in range(N):
                    u_d = plsc.load_gather(nblocks_v, [nrow[n], col])
                    new.append(accs[1 + n] + v_d * u_d)
                return tuple(new)

            accs = lax.fori_loop(0, D, d_body, (zeros,) * (N + 1))
            loss = _log_sigmoid(jnp.clip(accs[0], -10.0, 10.0))
            for n in range(N):
                loss = loss + _log_sigmoid(-jnp.clip(accs[1 + n], -10.0, 10.0))
            out_v[pl.ds(g * 16, 16)] = -loss
            return gcarry

        lax.fori_loop(0, G, group_body, 0)
        pltpu.sync_copy(out_v, out_hbm.at[pl.ds(start, C)])
        return carry

    lax.fori_loop(0, BPW // C, chunk_body, 0)


TW = 8192
PW = TW // RPB
TGRID = (V + TW - 1) // TW
VBP = TGRID * PW


def _tr_body(x_ref, o_ref):
    x = x_ref[...]
    xcat = jnp.concatenate(
        [x[:, j * PW:(j + 1) * PW] for j in range(RPB)], axis=0)
    o_ref[...] = xcat.T


def _to_blocks(emb):
    return pl.pallas_call(
        _tr_body,
        grid=(TGRID,),
        in_specs=[pl.BlockSpec((D, TW), lambda i: (0, i))],
        out_specs=pl.BlockSpec((PW, RPB * D), lambda i: (i, 0)),
        out_shape=jax.ShapeDtypeStruct((VBP, RPB * D), jnp.float32),
    )(emb.T)


def kernel(center, outside, negative, emb_v, emb_u):
    vtab = _to_blocks(emb_v).reshape(VBP * RPB, D)
    utab = _to_blocks(emb_u).reshape(VBP * RPB, D)
    return _sc_loss(center, outside, negative.reshape(-1), vtab, utab)


# speedup vs baseline: 4.2164x; 1.2650x over previous
import functools

import jax
import jax.numpy as jnp
from jax import lax
from jax.experimental import pallas as pl
from jax.experimental.pallas import tpu as pltpu
from jax.experimental.pallas import tpu_sc as plsc

B = 16384
N = 20
D = 32
V = 1000000
RPB = 4
VB = V // RPB
NC = 2
NS = 16
NW = NC * NS
BPW = B // NW
C = 128
G = C // 16
NIDX = C * N

TW = 16384
TWB = TW.bit_length() - 1
PW = TW // RPB
PWB = PW.bit_length() - 1
TGRID = (V + TW - 1) // TW
VBP = TGRID * PW


def _log_sigmoid(x):
    z = jnp.exp(-jnp.abs(x))
    t = z / (z + 2.0)
    t2 = t * t
    p = 2.0 * t * (1.0 + t2 * (1.0 / 3.0 + t2 * (0.2 + t2 * (1.0 / 7.0 + t2 * (1.0 / 9.0)))))
    return jnp.minimum(x, 0.0) - p


@functools.partial(
    pl.kernel,
    mesh=plsc.VectorSubcoreMesh(core_axis_name="c", subcore_axis_name="s"),
    compiler_params=pltpu.CompilerParams(
        needs_layout_passes=False, use_tc_tiling_on_sc=False),
    out_type=jax.ShapeDtypeStruct((B,), jnp.float32),
    scratch_types=[
        pltpu.VMEM((C,), jnp.int32),
        pltpu.VMEM((C,), jnp.int32),
        pltpu.VMEM((NIDX,), jnp.int32),
        pltpu.VMEM((C,), jnp.int32),
        pltpu.VMEM((C,), jnp.int32),
        pltpu.VMEM((NIDX,), jnp.int32),
        pltpu.VMEM((C, D), jnp.float32),
        pltpu.VMEM((C, D), jnp.float32),
        pltpu.VMEM((NIDX, D), jnp.float32),
        pltpu.VMEM((C,), jnp.float32),
        pltpu.SemaphoreType.DMA,
    ],
)
def _sc_loss(center_hbm, outside_hbm, negflat_hbm, vtab_hbm, utab_hbm,
             out_hbm, cidx_v, oidx_v, nidx_v, cblk_v, oblk_v, nblk_v,
             vblocks_v, oblocks_v, nblocks_v, out_v, sem):
    wid = lax.axis_index("s") * NC + lax.axis_index("c")
    base = wid * BPW

    def chunk_body(c, carry):
        start = base + c * C
        pltpu.sync_copy(center_hbm.at[pl.ds(start, C)], cidx_v)
        pltpu.sync_copy(outside_hbm.at[pl.ds(start, C)], oidx_v)
        pltpu.sync_copy(negflat_hbm.at[pl.ds(start * N, NIDX)], nidx_v)
        def _blk(r):
            return ((r >> TWB) << TWB) | ((r & (PW - 1)) << 2) | ((r >> PWB) & 3)

        for i in range(C // 16):
            cblk_v[pl.ds(i * 16, 16)] = _blk(cidx_v[pl.ds(i * 16, 16)])
            oblk_v[pl.ds(i * 16, 16)] = _blk(oidx_v[pl.ds(i * 16, 16)])
        for i in range(NIDX // 16):
            nblk_v[pl.ds(i * 16, 16)] = _blk(nidx_v[pl.ds(i * 16, 16)])
        copies = [
            pltpu.async_copy(vtab_hbm.at[cblk_v], vblocks_v, sem),
            pltpu.async_copy(utab_hbm.at[oblk_v], oblocks_v, sem),
        ]
        for j in range(NIDX // 128):
            copies.append(pltpu.async_copy(
                utab_hbm.at[nblk_v.at[pl.ds(j * 128, 128)]],
                nblocks_v.at[pl.ds(j * 128, 128)], sem))
        for cp in copies:
            cp.wait()

        def group_body(g, gcarry):
            zeros = jnp.zeros((16,), jnp.float32)
            lanes = lax.iota(jnp.int32, 16)
            rows16 = g * 16 + lanes
            nrow = [rows16 * N + n for n in range(N)]

            def d_body(dd, accs):
                col = (dd + lanes) & (D - 1)
                v_d = plsc.load_gather(vblocks_v, [rows16, col])
                o_d = plsc.load_gather(oblocks_v, [rows16, col])
                new = [accs[0] + v_d * o_d]
                for n in range(N):
                    u_d = plsc.load_gather(nblocks_v, [nrow[n], col])
                    new.append(accs[1 + n] + v_d * u_d)
                return tuple(new)

            accs = lax.fori_loop(0, D, d_body, (zeros,) * (N + 1))
            loss = _log_sigmoid(jnp.clip(accs[0], -10.0, 10.0))
            for n in range(N):
                loss = loss + _log_sigmoid(-jnp.clip(accs[1 + n], -10.0, 10.0))
            out_v[pl.ds(g * 16, 16)] = -loss
            return gcarry

        lax.fori_loop(0, G, group_body, 0)
        pltpu.sync_copy(out_v, out_hbm.at[pl.ds(start, C)])
        return carry

    lax.fori_loop(0, BPW // C, chunk_body, 0)


def _tr_body(x_ref, o_ref):
    x = x_ref[...]
    xcat = jnp.concatenate(
        [x[:, j * PW:(j + 1) * PW] for j in range(RPB)], axis=0)
    o_ref[...] = xcat.T


def _to_blocks(emb):
    return pl.pallas_call(
        _tr_body,
        grid=(TGRID,),
        in_specs=[pl.BlockSpec((D, TW), lambda i: (0, i))],
        out_specs=pl.BlockSpec((PW, RPB * D), lambda i: (i, 0)),
        out_shape=jax.ShapeDtypeStruct((VBP, RPB * D), jnp.float32),
    )(emb.T)


def kernel(center, outside, negative, emb_v, emb_u):
    vtab = _to_blocks(emb_v).reshape(VBP * RPB, D)
    utab = _to_blocks(emb_u).reshape(VBP * RPB, D)
    return _sc_loss(center, outside, negative.reshape(-1), vtab, utab)


# speedup vs baseline: 4.6856x; 1.1113x over previous
import functools

import jax
import jax.numpy as jnp
from jax import lax
from jax.experimental import pallas as pl
from jax.experimental.pallas import tpu as pltpu
from jax.experimental.pallas import tpu_sc as plsc

B = 16384
N = 20
D = 32
V = 1000000
RPB = 4
VB = V // RPB
NC = 2
NS = 16
NW = NC * NS
BPW = B // NW
C = 128
G = C // 16
NIDX = C * N

TW = 32768
TWB = TW.bit_length() - 1
PW = TW // RPB
PWB = PW.bit_length() - 1
TGRID = (V + TW - 1) // TW
VBP = TGRID * PW


def _log_sigmoid(x):
    z = jnp.exp(-jnp.abs(x))
    t = z / (z + 2.0)
    t2 = t * t
    p = 2.0 * t * (1.0 + t2 * (1.0 / 3.0 + t2 * (0.2 + t2 * (1.0 / 7.0 + t2 * (1.0 / 9.0)))))
    return jnp.minimum(x, 0.0) - p


@functools.partial(
    pl.kernel,
    mesh=plsc.VectorSubcoreMesh(core_axis_name="c", subcore_axis_name="s"),
    compiler_params=pltpu.CompilerParams(
        needs_layout_passes=False, use_tc_tiling_on_sc=False),
    out_type=jax.ShapeDtypeStruct((B,), jnp.float32),
    scratch_types=[
        pltpu.VMEM((C,), jnp.int32),
        pltpu.VMEM((C,), jnp.int32),
        pltpu.VMEM((NIDX,), jnp.int32),
        pltpu.VMEM((C,), jnp.int32),
        pltpu.VMEM((C,), jnp.int32),
        pltpu.VMEM((NIDX,), jnp.int32),
        pltpu.VMEM((C, D), jnp.float32),
        pltpu.VMEM((C, D), jnp.float32),
        pltpu.VMEM((NIDX, D), jnp.float32),
        pltpu.VMEM((C,), jnp.float32),
        pltpu.SemaphoreType.DMA,
    ],
)
def _sc_loss(center_hbm, outside_hbm, negflat_hbm, vtab_hbm, utab_hbm,
             out_hbm, cidx_v, oidx_v, nidx_v, cblk_v, oblk_v, nblk_v,
             vblocks_v, oblocks_v, nblocks_v, out_v, sem):
    wid = lax.axis_index("s") * NC + lax.axis_index("c")
    base = wid * BPW

    def chunk_body(c, carry):
        start = base + c * C
        pltpu.sync_copy(center_hbm.at[pl.ds(start, C)], cidx_v)
        pltpu.sync_copy(outside_hbm.at[pl.ds(start, C)], oidx_v)
        pltpu.sync_copy(negflat_hbm.at[pl.ds(start * N, NIDX)], nidx_v)
        def _blk(r):
            return ((r >> TWB) << TWB) | ((r & (PW - 1)) << 2) | ((r >> PWB) & 3)

        for i in range(C // 16):
            cblk_v[pl.ds(i * 16, 16)] = _blk(cidx_v[pl.ds(i * 16, 16)])
            oblk_v[pl.ds(i * 16, 16)] = _blk(oidx_v[pl.ds(i * 16, 16)])
        for i in range(NIDX // 16):
            nblk_v[pl.ds(i * 16, 16)] = _blk(nidx_v[pl.ds(i * 16, 16)])
        copies = [
            pltpu.async_copy(vtab_hbm.at[cblk_v], vblocks_v, sem),
            pltpu.async_copy(utab_hbm.at[oblk_v], oblocks_v, sem),
        ]
        for j in range(NIDX // 128):
            copies.append(pltpu.async_copy(
                utab_hbm.at[nblk_v.at[pl.ds(j * 128, 128)]],
                nblocks_v.at[pl.ds(j * 128, 128)], sem))
        for cp in copies:
            cp.wait()

        def group_body(g, gcarry):
            zeros = jnp.zeros((16,), jnp.float32)
            lanes = lax.iota(jnp.int32, 16)
            rows16 = g * 16 + lanes
            nrow = [rows16 * N + n for n in range(N)]

            def d_body(dd, accs):
                col = (dd + lanes) & (D - 1)
                v_d = plsc.load_gather(vblocks_v, [rows16, col])
                o_d = plsc.load_gather(oblocks_v, [rows16, col])
                new = [accs[0] + v_d * o_d]
                for n in range(N):
                    u_d = plsc.load_gather(nblocks_v, [nrow[n], col])
                    new.append(accs[1 + n] + v_d * u_d)
                return tuple(new)

            accs = lax.fori_loop(0, D, d_body, (zeros,) * (N + 1))
            loss = _log_sigmoid(jnp.clip(accs[0], -10.0, 10.0))
            for n in range(N):
                loss = loss + _log_sigmoid(-jnp.clip(accs[1 + n], -10.0, 10.0))
            out_v[pl.ds(g * 16, 16)] = -loss
            return gcarry

        lax.fori_loop(0, G, group_body, 0)
        pltpu.sync_copy(out_v, out_hbm.at[pl.ds(start, C)])
        return carry

    lax.fori_loop(0, BPW // C, chunk_body, 0)


def _tr_body(x_ref, o_ref):
    x = x_ref[...]
    xcat = jnp.concatenate(
        [x[:, j * PW:(j + 1) * PW] for j in range(RPB)], axis=0)
    o_ref[...] = xcat.T


def _to_blocks(emb):
    return pl.pallas_call(
        _tr_body,
        grid=(TGRID,),
        in_specs=[pl.BlockSpec((D, TW), lambda i: (0, i))],
        out_specs=pl.BlockSpec((PW, RPB * D), lambda i: (i, 0)),
        out_shape=jax.ShapeDtypeStruct((VBP, RPB * D), jnp.float32),
    )(emb.T)


def kernel(center, outside, negative, emb_v, emb_u):
    vtab = _to_blocks(emb_v).reshape(VBP * RPB, D)
    utab = _to_blocks(emb_u).reshape(VBP * RPB, D)
    return _sc_loss(center, outside, negative.reshape(-1), vtab, utab)


# speedup vs baseline: 4.7445x; 1.0126x over previous
import functools

import jax
import jax.numpy as jnp
from jax import lax
from jax.experimental import pallas as pl
from jax.experimental.pallas import tpu as pltpu
from jax.experimental.pallas import tpu_sc as plsc

B = 16384
N = 20
D = 32
V = 1000000
RPB = 4
VB = V // RPB
NC = 2
NS = 16
NW = NC * NS
BPW = B // NW
C = 128
G = C // 16
NIDX = C * N

TW = 65536
TWB = TW.bit_length() - 1
PW = TW // RPB
PWB = PW.bit_length() - 1
TGRID = (V + TW - 1) // TW
VBP = TGRID * PW


def _log_sigmoid(x):
    z = jnp.exp(-jnp.abs(x))
    t = z / (z + 2.0)
    t2 = t * t
    p = 2.0 * t * (1.0 + t2 * (1.0 / 3.0 + t2 * (0.2 + t2 * (1.0 / 7.0 + t2 * (1.0 / 9.0)))))
    return jnp.minimum(x, 0.0) - p


@functools.partial(
    pl.kernel,
    mesh=plsc.VectorSubcoreMesh(core_axis_name="c", subcore_axis_name="s"),
    compiler_params=pltpu.CompilerParams(
        needs_layout_passes=False, use_tc_tiling_on_sc=False),
    out_type=jax.ShapeDtypeStruct((B,), jnp.float32),
    scratch_types=[
        pltpu.VMEM((C,), jnp.int32),
        pltpu.VMEM((C,), jnp.int32),
        pltpu.VMEM((NIDX,), jnp.int32),
        pltpu.VMEM((C,), jnp.int32),
        pltpu.VMEM((C,), jnp.int32),
        pltpu.VMEM((NIDX,), jnp.int32),
        pltpu.VMEM((C, D), jnp.float32),
        pltpu.VMEM((C, D), jnp.float32),
        pltpu.VMEM((NIDX, D), jnp.float32),
        pltpu.VMEM((C,), jnp.float32),
        pltpu.SemaphoreType.DMA,
    ],
)
def _sc_loss(center_hbm, outside_hbm, negflat_hbm, vtab_hbm, utab_hbm,
             out_hbm, cidx_v, oidx_v, nidx_v, cblk_v, oblk_v, nblk_v,
             vblocks_v, oblocks_v, nblocks_v, out_v, sem):
    wid = lax.axis_index("s") * NC + lax.axis_index("c")
    base = wid * BPW

    def chunk_body(c, carry):
        start = base + c * C
        pltpu.sync_copy(center_hbm.at[pl.ds(start, C)], cidx_v)
        pltpu.sync_copy(outside_hbm.at[pl.ds(start, C)], oidx_v)
        pltpu.sync_copy(negflat_hbm.at[pl.ds(start * N, NIDX)], nidx_v)
        def _blk(r):
            return ((r >> TWB) << TWB) | ((r & (PW - 1)) << 2) | ((r >> PWB) & 3)

        for i in range(C // 16):
            cblk_v[pl.ds(i * 16, 16)] = _blk(cidx_v[pl.ds(i * 16, 16)])
            oblk_v[pl.ds(i * 16, 16)] = _blk(oidx_v[pl.ds(i * 16, 16)])
        for i in range(NIDX // 16):
            nblk_v[pl.ds(i * 16, 16)] = _blk(nidx_v[pl.ds(i * 16, 16)])
        copies = [
            pltpu.async_copy(vtab_hbm.at[cblk_v], vblocks_v, sem),
            pltpu.async_copy(utab_hbm.at[oblk_v], oblocks_v, sem),
        ]
        for j in range(NIDX // 128):
            copies.append(pltpu.async_copy(
                utab_hbm.at[nblk_v.at[pl.ds(j * 128, 128)]],
                nblocks_v.at[pl.ds(j * 128, 128)], sem))
        for cp in copies:
            cp.wait()

        def group_body(g, gcarry):
            zeros = jnp.zeros((16,), jnp.float32)
            lanes = lax.iota(jnp.int32, 16)
            rows16 = g * 16 + lanes
            nrow = [rows16 * N + n for n in range(N)]

            def d_body(dd, accs):
                col = (dd + lanes) & (D - 1)
                v_d = plsc.load_gather(vblocks_v, [rows16, col])
                o_d = plsc.load_gather(oblocks_v, [rows16, col])
                new = [accs[0] + v_d * o_d]
                for n in range(N):
                    u_d = plsc.load_gather(nblocks_v, [nrow[n], col])
                    new.append(accs[1 + n] + v_d * u_d)
                return tuple(new)

            accs = lax.fori_loop(0, D, d_body, (zeros,) * (N + 1))
            loss = _log_sigmoid(jnp.clip(accs[0], -10.0, 10.0))
            for n in range(N):
                loss = loss + _log_sigmoid(-jnp.clip(accs[1 + n], -10.0, 10.0))
            out_v[pl.ds(g * 16, 16)] = -loss
            return gcarry

        lax.fori_loop(0, G, group_body, 0)
        pltpu.sync_copy(out_v, out_hbm.at[pl.ds(start, C)])
        return carry

    lax.fori_loop(0, BPW // C, chunk_body, 0)


def _tr_body(x_ref, o_ref):
    x = x_ref[...]
    xcat = jnp.concatenate(
        [x[:, j * PW:(j + 1) * PW] for j in range(RPB)], axis=0)
    o_ref[...] = xcat.T


def _to_blocks(emb):
    return pl.pallas_call(
        _tr_body,
        grid=(TGRID,),
        in_specs=[pl.BlockSpec((D, TW), lambda i: (0, i))],
        out_specs=pl.BlockSpec((PW, RPB * D), lambda i: (i, 0)),
        out_shape=jax.ShapeDtypeStruct((VBP, RPB * D), jnp.float32),
    )(emb.T)


def kernel(center, outside, negative, emb_v, emb_u):
    vtab = _to_blocks(emb_v).reshape(VBP * RPB, D)
    utab = _to_blocks(emb_u).reshape(VBP * RPB, D)
    return _sc_loss(center, outside, negative.reshape(-1), vtab, utab)
